# baseline (speedup 1.0000x reference)
import functools

import jax
import jax.numpy as jnp
from jax import lax
from jax.experimental import pallas as pl
from jax.experimental.pallas import tpu as pltpu
from jax.experimental.pallas import tpu_sc as plsc

N = 10000
NP = 10240
E = 320000
D = 128

NC = 2
NS = 16
NW = NC * NS
CH = 128
NCHT = 80
E_PAD = NW * NCHT * CH
ROWS_PT = NP // NS

_MESH = plsc.VectorSubcoreMesh(core_axis_name="c", subcore_axis_name="s")


def _deg_body(dstm, ewm, z_hbm, out_hbm, dst_v, ew_v, deg_sh):
    cid = lax.axis_index("c")
    sid = lax.axis_index("s")
    wid = sid * NC + cid
    pltpu.sync_copy(z_hbm.at[pl.ds(sid * ROWS_PT, ROWS_PT)],
                    deg_sh.at[pl.ds(sid * ROWS_PT, ROWS_PT)])
    pltpu.sync_copy(dstm.at[wid], dst_v)
    pltpu.sync_copy(ewm.at[wid], ew_v)
    plsc.subcore_barrier()

    def chunk(j, carry):
        pltpu.sync_copy(ew_v.at[j], deg_sh.at[dst_v.at[j]], add=True)
        return carry

    lax.fori_loop(0, NCHT, chunk, 0)
    plsc.subcore_barrier()
    pltpu.sync_copy(deg_sh.at[pl.ds(sid * ROWS_PT, ROWS_PT)],
                    out_hbm.at[cid, pl.ds(sid * ROWS_PT, ROWS_PT)])


_deg_kernel = pl.kernel(
    _deg_body, mesh=_MESH,
    out_type=jax.ShapeDtypeStruct((NC, NP), jnp.float32),
    scratch_types=[
        pltpu.VMEM((NCHT, CH), jnp.int32),
        pltpu.VMEM((NCHT, CH), jnp.float32),
        pltpu.VMEM_SHARED((NP,), jnp.float32),
    ],
)


G = 10
NWIN = NCHT // G


def _agg_body(h_hbm, comb_hbm, z_hbm, out_hbm,
              w0, w1, r0, r1, acc_sh,
              wsem0, wsem1, gsem0, gsem1, ssem0, ssem1):
    cid = lax.axis_index("c")
    sid = lax.axis_index("s")
    wid = sid * NC + cid
    winb = [w0, w1]
    rows = [r0, r1]
    wsem = [wsem0, wsem1]
    gsem = [gsem0, gsem1]
    ssem = [ssem0, ssem1]

    pltpu.sync_copy(z_hbm.at[pl.ds(sid * ROWS_PT, ROWS_PT)],
                    acc_sh.at[pl.ds(sid * ROWS_PT, ROWS_PT)])
    pltpu.sync_copy(comb_hbm.at[wid, 0], winb[0])
    plsc.subcore_barrier()
    pltpu.async_copy(h_hbm.at[winb[0].at[0]], rows[0], gsem[0])

    def window_pair(w2, carry):
        for p in range(2):
            w = 2 * w2 + p
            wb = winb[p]
            nwb = winb[1 - p]

            def phase_pair(t, c2):
                for b in range(2):
                    jj = 2 * t + b
                    buf = rows[b]
                    nb = 1 - b
                    pltpu.make_async_copy(
                        h_hbm.at[wb.at[jj]], buf, gsem[b]).wait()

                    def edge_group(g, c3):
                        ews_i = wb[2 * G + jj, pl.ds(g * 16, 16)]
                        ews = lax.bitcast_convert_type(ews_i, jnp.float32)
                        for i in range(16):
                            sc = ews[i]
                            e = g * 16 + i
                            for k in range(D // 16):
                                sl = pl.ds(k * 16, 16)
                                buf[e, sl] = buf[e, sl] * sc
                        return c3
                    lax.fori_loop(0, CH // 16, edge_group, 0)

                    def wait_sc():
                        pltpu.make_async_copy(
                            rows[nb], acc_sh.at[wb.at[G]], ssem[nb]).wait()
                    if p == 0 and b == 0:
                        pl.when((w2 > 0) | (t > 0))(wait_sc)
                    else:
                        wait_sc()

                    if b == 0:
                        def load_next():
                            pltpu.async_copy(
                                comb_hbm.at[wid, w + 1], nwb, wsem[1 - p])
                        cond = t == 0
                        if p == 1:
                            cond = cond & (w2 < NWIN // 2 - 1)
                        pl.when(cond)(load_next)

                    pltpu.async_copy(
                        buf, acc_sh.at[wb.at[G + jj]], ssem[b], add=True)

                    def g_in_window():
                        pltpu.async_copy(
                            h_hbm.at[wb.at[jj + 1]], rows[nb], gsem[nb])

                    def g_boundary():
                        pltpu.make_async_copy(
                            comb_hbm.at[wid, 0], nwb, wsem[1 - p]).wait()
                        pltpu.async_copy(
                            h_hbm.at[nwb.at[0]], rows[nb], gsem[nb])
                    if b == 0:
                        g_in_window()
                    else:
                        pl.when(t < G // 2 - 1)(g_in_window)
                        if p == 0:
                            pl.when(t == G // 2 - 1)(g_boundary)
                        else:
                            pl.when((t == G // 2 - 1) &
                                    (w2 < NWIN // 2 - 1))(g_boundary)
                return c2

            lax.fori_loop(0, G // 2, phase_pair, 0)
        return carry

    lax.fori_loop(0, NWIN // 2, window_pair, 0)
    pltpu.make_async_copy(
        rows[1], acc_sh.at[winb[1].at[G]], ssem[1]).wait()
    plsc.subcore_barrier()
    pltpu.sync_copy(acc_sh.at[pl.ds(sid * ROWS_PT, ROWS_PT)],
                    out_hbm.at[cid, pl.ds(sid * ROWS_PT, ROWS_PT)])


_agg_kernel = pl.kernel(
    _agg_body, mesh=_MESH,
    out_type=jax.ShapeDtypeStruct((NC, NP, D), jnp.float32),
    scratch_types=[
        pltpu.VMEM((3 * G, CH), jnp.int32),
        pltpu.VMEM((3 * G, CH), jnp.int32),
        pltpu.VMEM((CH, D), jnp.float32),
        pltpu.VMEM((CH, D), jnp.float32),
        pltpu.VMEM_SHARED((NP, D), jnp.float32),
        pltpu.SemaphoreType.DMA,
        pltpu.SemaphoreType.DMA,
        pltpu.SemaphoreType.DMA,
        pltpu.SemaphoreType.DMA,
        pltpu.SemaphoreType.DMA,
        pltpu.SemaphoreType.DMA,
    ],
)


BM = 1024


def _dis(d0_ref, d1_ref):
    return lax.rsqrt(d0_ref[...] + d1_ref[...] + 1.0)


def _mm1_body(x_ref, w_ref, d0_ref, d1_ref, o_ref):
    dis = _dis(d0_ref, d1_ref)
    o_ref[...] = dis * jnp.dot(x_ref[...], w_ref[...],
                               preferred_element_type=jnp.float32)


def _mid_body(a0_ref, a1_ref, hp_ref, d0_ref, d1_ref, b_ref, w_ref, o_ref):
    dis = _dis(d0_ref, d1_ref)
    t = dis * (a0_ref[...] + a1_ref[...] + hp_ref[...]) + b_ref[...]
    t = jnp.where(t >= 0.0, t, 0.01 * t)
    o_ref[...] = dis * jnp.dot(t, w_ref[...],
                               preferred_element_type=jnp.float32)


def _fin_body(a0_ref, a1_ref, hp_ref, d0_ref, d1_ref, b_ref, o_ref):
    dis = _dis(d0_ref, d1_ref)
    o_ref[...] = dis * (a0_ref[...] + a1_ref[...] + hp_ref[...]) + b_ref[...]


_nd_spec = pl.BlockSpec((BM, D), lambda i: (i, 0))
_d_spec = pl.BlockSpec((BM, 1), lambda i: (i, 0))
_w_spec = pl.BlockSpec((D, D), lambda i: (0, 0))
_b_spec = pl.BlockSpec((1, D), lambda i: (0, 0))
_out_nd = jax.ShapeDtypeStruct((NP, D), jnp.float32)
_grid = (NP // BM,)

_mm1 = pl.pallas_call(
    _mm1_body, grid=_grid,
    in_specs=[_nd_spec, _w_spec, _d_spec, _d_spec],
    out_specs=_nd_spec, out_shape=_out_nd)

_mid = pl.pallas_call(
    _mid_body, grid=_grid,
    in_specs=[_nd_spec, _nd_spec, _nd_spec, _d_spec, _d_spec, _b_spec, _w_spec],
    out_specs=_nd_spec, out_shape=_out_nd)

_fin = pl.pallas_call(
    _fin_body, grid=_grid,
    in_specs=[_nd_spec, _nd_spec, _nd_spec, _d_spec, _d_spec, _b_spec],
    out_specs=_nd_spec, out_shape=_out_nd)


def kernel(x, edge_index, edge_attr, W1, b1, W2, b2, W3, b3):
    src = edge_index[0].astype(jnp.int32)
    dst = edge_index[1].astype(jnp.int32)
    ew = edge_attr.astype(jnp.float32)

    pad = E_PAD - E
    spread = (jnp.arange(pad, dtype=jnp.int32) * 13) % N
    srcm = jnp.concatenate([src, spread])
    dstm = jnp.concatenate([dst, spread])
    ewm = jnp.concatenate([ew, jnp.zeros((pad,), jnp.float32)])
    srcm = srcm.reshape(NW, NCHT, CH)
    dstm = dstm.reshape(NW, NCHT, CH)
    ewm = ewm.reshape(NW, NCHT, CH)

    xpad = jnp.concatenate([x, jnp.zeros((NP - N, D), jnp.float32)])
    z_n1 = jnp.zeros((NP,), jnp.float32)
    z_nd = jnp.zeros((NP, D), jnp.float32)

    ew_i = lax.bitcast_convert_type(ewm, jnp.int32)
    comb = jnp.concatenate([
        srcm.reshape(NW, NWIN, G, CH),
        dstm.reshape(NW, NWIN, G, CH),
        ew_i.reshape(NW, NWIN, G, CH),
    ], axis=2)

    degp = _deg_kernel(dstm, ewm, z_n1)
    d0 = degp[0].reshape(NP, 1)
    d1 = degp[1].reshape(NP, 1)

    b1r = b1.reshape(1, D)
    b2r = b2.reshape(1, D)
    b3r = b3.reshape(1, D)

    hp = _mm1(xpad, W1, d0, d1)
    accp = _agg_kernel(hp, comb, z_nd)
    hp = _mid(accp[0], accp[1], hp, d0, d1, b1r, W2)
    accp = _agg_kernel(hp, comb, z_nd)
    hp = _mid(accp[0], accp[1], hp, d0, d1, b2r, W3)
    accp = _agg_kernel(hp, comb, z_nd)
    return _fin(accp[0], accp[1], hp, d0, d1, b3r)[:N]


# speedup vs baseline: 1.2375x; 1.2375x over previous
import functools

import jax
import jax.numpy as jnp
from jax import lax
from jax.experimental import pallas as pl
from jax.experimental.pallas import tpu as pltpu
from jax.experimental.pallas import tpu_sc as plsc

N = 10000
NP = 10240
E = 320000
D = 128

NC = 2
NS = 16
NW = NC * NS
CH = 128
NCHT = 80
E_PAD = NW * NCHT * CH
ROWS_PT = NP // NS

_MESH = plsc.VectorSubcoreMesh(core_axis_name="c", subcore_axis_name="s")


def _deg_body(dstm, ewm, z_hbm, out_hbm, dst_v, ew_v, deg_sh):
    cid = lax.axis_index("c")
    sid = lax.axis_index("s")
    wid = sid * NC + cid
    pltpu.sync_copy(z_hbm.at[pl.ds(sid * ROWS_PT, ROWS_PT)],
                    deg_sh.at[pl.ds(sid * ROWS_PT, ROWS_PT)])
    pltpu.sync_copy(dstm.at[wid], dst_v)
    pltpu.sync_copy(ewm.at[wid], ew_v)
    plsc.subcore_barrier()

    def chunk(j, carry):
        pltpu.sync_copy(ew_v.at[j], deg_sh.at[dst_v.at[j]], add=True)
        return carry

    lax.fori_loop(0, NCHT, chunk, 0)
    plsc.subcore_barrier()
    pltpu.sync_copy(deg_sh.at[pl.ds(sid * ROWS_PT, ROWS_PT)],
                    out_hbm.at[cid, pl.ds(sid * ROWS_PT, ROWS_PT)])


_deg_kernel = pl.kernel(
    _deg_body, mesh=_MESH,
    out_type=jax.ShapeDtypeStruct((NC, NP), jnp.float32),
    scratch_types=[
        pltpu.VMEM((NCHT, CH), jnp.int32),
        pltpu.VMEM((NCHT, CH), jnp.float32),
        pltpu.VMEM_SHARED((NP,), jnp.float32),
    ],
)


G = 10
NWIN = NCHT // G


def _agg_body(h_hbm, comb_hbm, z_hbm, out_hbm,
              w0, w1, r0, r1, acc_sh,
              wsem0, wsem1, gsem0, gsem1, ssem0, ssem1):
    cid = lax.axis_index("c")
    sid = lax.axis_index("s")
    wid = sid * NC + cid
    winb = [w0, w1]
    rows = [r0, r1]
    wsem = [wsem0, wsem1]
    gsem = [gsem0, gsem1]
    ssem = [ssem0, ssem1]

    pltpu.sync_copy(z_hbm.at[pl.ds(sid * ROWS_PT, ROWS_PT)],
                    acc_sh.at[pl.ds(sid * ROWS_PT, ROWS_PT)])
    pltpu.sync_copy(comb_hbm.at[wid, 0], winb[0])
    plsc.subcore_barrier()
    pltpu.async_copy(h_hbm.at[winb[0].at[0]], rows[0], gsem[0])

    def window_pair(w2, carry):
        for p in range(2):
            w = 2 * w2 + p
            wb = winb[p]
            nwb = winb[1 - p]

            def phase_pair(t, c2):
                for b in range(2):
                    jj = 2 * t + b
                    buf = rows[b]
                    nb = 1 - b
                    pltpu.make_async_copy(
                        h_hbm.at[wb.at[jj]], buf, gsem[b]).wait()

                    if b == 0:
                        def load_next():
                            pltpu.async_copy(
                                comb_hbm.at[wid, w + 1], nwb, wsem[1 - p])
                        cond = t == 0
                        if p == 1:
                            cond = cond & (w2 < NWIN // 2 - 1)
                        pl.when(cond)(load_next)

                    def g_in_window():
                        pltpu.async_copy(
                            h_hbm.at[wb.at[jj + 1]], rows[nb], gsem[nb])

                    def g_boundary():
                        pltpu.make_async_copy(
                            comb_hbm.at[wid, 0], nwb, wsem[1 - p]).wait()
                        pltpu.async_copy(
                            h_hbm.at[nwb.at[0]], rows[nb], gsem[nb])
                    if b == 0:
                        g_in_window()
                    else:
                        pl.when(t < G // 2 - 1)(g_in_window)
                        if p == 0:
                            pl.when(t == G // 2 - 1)(g_boundary)
                        else:
                            pl.when((t == G // 2 - 1) &
                                    (w2 < NWIN // 2 - 1))(g_boundary)

                    def edge_group(g, c3):
                        ews_i = wb[2 * G + jj, pl.ds(g * 16, 16)]
                        ews = lax.bitcast_convert_type(ews_i, jnp.float32)
                        for i in range(16):
                            sc = ews[i]
                            e = g * 16 + i
                            for k in range(D // 16):
                                sl = pl.ds(k * 16, 16)
                                buf[e, sl] = buf[e, sl] * sc
                        return c3
                    lax.fori_loop(0, CH // 16, edge_group, 0)

                    pltpu.sync_copy(buf, acc_sh.at[wb.at[G + jj]], add=True)
                return c2

            lax.fori_loop(0, G // 2, phase_pair, 0)
        return carry

    lax.fori_loop(0, NWIN // 2, window_pair, 0)
    plsc.subcore_barrier()
    pltpu.sync_copy(acc_sh.at[pl.ds(sid * ROWS_PT, ROWS_PT)],
                    out_hbm.at[cid, pl.ds(sid * ROWS_PT, ROWS_PT)])


_agg_kernel = pl.kernel(
    _agg_body, mesh=_MESH,
    out_type=jax.ShapeDtypeStruct((NC, NP, D), jnp.float32),
    scratch_types=[
        pltpu.VMEM((3 * G, CH), jnp.int32),
        pltpu.VMEM((3 * G, CH), jnp.int32),
        pltpu.VMEM((CH, D), jnp.float32),
        pltpu.VMEM((CH, D), jnp.float32),
        pltpu.VMEM_SHARED((NP, D), jnp.float32),
        pltpu.SemaphoreType.DMA,
        pltpu.SemaphoreType.DMA,
        pltpu.SemaphoreType.DMA,
        pltpu.SemaphoreType.DMA,
        pltpu.SemaphoreType.DMA,
        pltpu.SemaphoreType.DMA,
    ],
)


BM = 1024


def _dis(d0_ref, d1_ref):
    return lax.rsqrt(d0_ref[...] + d1_ref[...] + 1.0)


def _mm1_body(x_ref, w_ref, d0_ref, d1_ref, o_ref):
    dis = _dis(d0_ref, d1_ref)
    o_ref[...] = dis * jnp.dot(x_ref[...], w_ref[...],
                               preferred_element_type=jnp.float32)


def _mid_body(a0_ref, a1_ref, hp_ref, d0_ref, d1_ref, b_ref, w_ref, o_ref):
    dis = _dis(d0_ref, d1_ref)
    t = dis * (a0_ref[...] + a1_ref[...] + hp_ref[...]) + b_ref[...]
    t = jnp.where(t >= 0.0, t, 0.01 * t)
    o_ref[...] = dis * jnp.dot(t, w_ref[...],
                               preferred_element_type=jnp.float32)


def _fin_body(a0_ref, a1_ref, hp_ref, d0_ref, d1_ref, b_ref, o_ref):
    dis = _dis(d0_ref, d1_ref)
    o_ref[...] = dis * (a0_ref[...] + a1_ref[...] + hp_ref[...]) + b_ref[...]


_nd_spec = pl.BlockSpec((BM, D), lambda i: (i, 0))
_d_spec = pl.BlockSpec((BM, 1), lambda i: (i, 0))
_w_spec = pl.BlockSpec((D, D), lambda i: (0, 0))
_b_spec = pl.BlockSpec((1, D), lambda i: (0, 0))
_out_nd = jax.ShapeDtypeStruct((NP, D), jnp.float32)
_grid = (NP // BM,)

_mm1 = pl.pallas_call(
    _mm1_body, grid=_grid,
    in_specs=[_nd_spec, _w_spec, _d_spec, _d_spec],
    out_specs=_nd_spec, out_shape=_out_nd)

_mid = pl.pallas_call(
    _mid_body, grid=_grid,
    in_specs=[_nd_spec, _nd_spec, _nd_spec, _d_spec, _d_spec, _b_spec, _w_spec],
    out_specs=_nd_spec, out_shape=_out_nd)

_fin = pl.pallas_call(
    _fin_body, grid=_grid,
    in_specs=[_nd_spec, _nd_spec, _nd_spec, _d_spec, _d_spec, _b_spec],
    out_specs=_nd_spec, out_shape=_out_nd)


def kernel(x, edge_index, edge_attr, W1, b1, W2, b2, W3, b3):
    src = edge_index[0].astype(jnp.int32)
    dst = edge_index[1].astype(jnp.int32)
    ew = edge_attr.astype(jnp.float32)

    pad = E_PAD - E
    spread = (jnp.arange(pad, dtype=jnp.int32) * 13) % N
    srcm = jnp.concatenate([src, spread])
    dstm = jnp.concatenate([dst, spread])
    ewm = jnp.concatenate([ew, jnp.zeros((pad,), jnp.float32)])
    srcm = srcm.reshape(NW, NCHT, CH)
    dstm = dstm.reshape(NW, NCHT, CH)
    ewm = ewm.reshape(NW, NCHT, CH)

    xpad = jnp.concatenate([x, jnp.zeros((NP - N, D), jnp.float32)])
    z_n1 = jnp.zeros((NP,), jnp.float32)
    z_nd = jnp.zeros((NP, D), jnp.float32)

    ew_i = lax.bitcast_convert_type(ewm, jnp.int32)
    comb = jnp.concatenate([
        srcm.reshape(NW, NWIN, G, CH),
        dstm.reshape(NW, NWIN, G, CH),
        ew_i.reshape(NW, NWIN, G, CH),
    ], axis=2)

    degp = _deg_kernel(dstm, ewm, z_n1)
    d0 = degp[0].reshape(NP, 1)
    d1 = degp[1].reshape(NP, 1)

    b1r = b1.reshape(1, D)
    b2r = b2.reshape(1, D)
    b3r = b3.reshape(1, D)

    hp = _mm1(xpad, W1, d0, d1)
    accp = _agg_kernel(hp, comb, z_nd)
    hp = _mid(accp[0], accp[1], hp, d0, d1, b1r, W2)
    accp = _agg_kernel(hp, comb, z_nd)
    hp = _mid(accp[0], accp[1], hp, d0, d1, b2r, W3)
    accp = _agg_kernel(hp, comb, z_nd)
    return _fin(accp[0], accp[1], hp, d0, d1, b3r)[:N]


# speedup vs baseline: 1.2442x; 1.0055x over previous
import functools

import jax
import jax.numpy as jnp
from jax import lax
from jax.experimental import pallas as pl
from jax.experimental.pallas import tpu as pltpu
from jax.experimental.pallas import tpu_sc as plsc

N = 10000
NP = 10240
E = 320000
D = 128

NC = 2
NS = 16
NW = NC * NS
CH = 128
NCHT = 80
E_PAD = NW * NCHT * CH
ROWS_PT = NP // NS

_MESH = plsc.VectorSubcoreMesh(core_axis_name="c", subcore_axis_name="s")


def _deg_body(dstm, ewm, z_hbm, out_hbm, dst_v, ew_v, deg_sh):
    cid = lax.axis_index("c")
    sid = lax.axis_index("s")
    wid = sid * NC + cid
    pltpu.sync_copy(z_hbm.at[pl.ds(sid * ROWS_PT, ROWS_PT)],
                    deg_sh.at[pl.ds(sid * ROWS_PT, ROWS_PT)])
    pltpu.sync_copy(dstm.at[wid], dst_v)
    pltpu.sync_copy(ewm.at[wid], ew_v)
    plsc.subcore_barrier()

    def chunk(j, carry):
        pltpu.sync_copy(ew_v.at[j], deg_sh.at[dst_v.at[j]], add=True)
        return carry

    lax.fori_loop(0, NCHT, chunk, 0)
    plsc.subcore_barrier()
    pltpu.sync_copy(deg_sh.at[pl.ds(sid * ROWS_PT, ROWS_PT)],
                    out_hbm.at[cid, pl.ds(sid * ROWS_PT, ROWS_PT)])


_deg_kernel = pl.kernel(
    _deg_body, mesh=_MESH,
    out_type=jax.ShapeDtypeStruct((NC, NP), jnp.float32),
    scratch_types=[
        pltpu.VMEM((NCHT, CH), jnp.int32),
        pltpu.VMEM((NCHT, CH), jnp.float32),
        pltpu.VMEM_SHARED((NP,), jnp.float32),
    ],
)


G = 10
NWIN = NCHT // G


def _agg_body(h_hbm, comb_hbm, z_hbm, out_hbm,
              w0, w1, r0, r1, acc_sh,
              wsem0, wsem1, gsem0, gsem1, ssem0, ssem1):
    cid = lax.axis_index("c")
    sid = lax.axis_index("s")
    wid = sid * NC + cid
    winb = [w0, w1]
    rows = [r0, r1]
    wsem = [wsem0, wsem1]
    gsem = [gsem0, gsem1]
    ssem = [ssem0, ssem1]

    pltpu.sync_copy(z_hbm.at[pl.ds(sid * ROWS_PT, ROWS_PT)],
                    acc_sh.at[pl.ds(sid * ROWS_PT, ROWS_PT)])
    pltpu.sync_copy(comb_hbm.at[wid, 0], winb[0])
    plsc.subcore_barrier()
    pltpu.async_copy(h_hbm.at[winb[0].at[0]], rows[0], gsem[0])

    def window_pair(w2, carry):
        for p in range(2):
            w = 2 * w2 + p
            wb = winb[p]
            nwb = winb[1 - p]

            def phase_pair(t, c2):
                for b in range(2):
                    jj = 2 * t + b
                    buf = rows[b]
                    nb = 1 - b
                    pltpu.make_async_copy(
                        h_hbm.at[wb.at[jj]], buf, gsem[b]).wait()

                    if b == 0:
                        def load_next():
                            pltpu.async_copy(
                                comb_hbm.at[wid, w + 1], nwb, wsem[1 - p])
                        cond = t == 0
                        if p == 1:
                            cond = cond & (w2 < NWIN // 2 - 1)
                        pl.when(cond)(load_next)

                    def g_in_window():
                        pltpu.async_copy(
                            h_hbm.at[wb.at[jj + 1]], rows[nb], gsem[nb])

                    def g_boundary():
                        pltpu.make_async_copy(
                            comb_hbm.at[wid, 0], nwb, wsem[1 - p]).wait()
                        pltpu.async_copy(
                            h_hbm.at[nwb.at[0]], rows[nb], gsem[nb])
                    if b == 0:
                        g_in_window()
                    else:
                        pl.when(t < G // 2 - 1)(g_in_window)
                        if p == 0:
                            pl.when(t == G // 2 - 1)(g_boundary)
                        else:
                            pl.when((t == G // 2 - 1) &
                                    (w2 < NWIN // 2 - 1))(g_boundary)

                    def edge_group(g, c3):
                        ews_i = wb[2 * G + jj, pl.ds(g * 16, 16)]
                        ews = lax.bitcast_convert_type(ews_i, jnp.float32)
                        for i in range(16):
                            sc = ews[i]
                            e = g * 16 + i
                            for k in range(D // 16):
                                sl = pl.ds(k * 16, 16)
                                buf[e, sl] = buf[e, sl] * sc
                        return c3
                    lax.fori_loop(0, CH // 16, edge_group, 0)

                    pltpu.sync_copy(buf, acc_sh.at[wb.at[G + jj]], add=True)
                return c2

            lax.fori_loop(0, G // 2, phase_pair, 0)
        return carry

    lax.fori_loop(0, NWIN // 2, window_pair, 0)
    plsc.subcore_barrier()
    pltpu.sync_copy(acc_sh.at[pl.ds(sid * ROWS_PT, ROWS_PT)],
                    out_hbm.at[cid, pl.ds(sid * ROWS_PT, ROWS_PT)])


_agg_kernel = pl.kernel(
    _agg_body, mesh=_MESH,
    out_type=jax.ShapeDtypeStruct((NC, NP, D), jnp.float32),
    scratch_types=[
        pltpu.VMEM((3 * G, CH), jnp.int32),
        pltpu.VMEM((3 * G, CH), jnp.int32),
        pltpu.VMEM((CH, D), jnp.float32),
        pltpu.VMEM((CH, D), jnp.float32),
        pltpu.VMEM_SHARED((NP, D), jnp.float32),
        pltpu.SemaphoreType.DMA,
        pltpu.SemaphoreType.DMA,
        pltpu.SemaphoreType.DMA,
        pltpu.SemaphoreType.DMA,
        pltpu.SemaphoreType.DMA,
        pltpu.SemaphoreType.DMA,
    ],
)


BM = 1024


def _dis(d0_ref, d1_ref):
    return lax.rsqrt(d0_ref[...] + d1_ref[...] + 1.0)


def _mm1_body(x_ref, w_ref, d0_ref, d1_ref, o_ref):
    dis = _dis(d0_ref, d1_ref)
    o_ref[...] = dis * jnp.dot(x_ref[...], w_ref[...],
                               preferred_element_type=jnp.float32)


def _mid_body(a0_ref, a1_ref, hp_ref, d0_ref, d1_ref, b_ref, w_ref, o_ref):
    dis = _dis(d0_ref, d1_ref)
    t = dis * (a0_ref[...] + a1_ref[...] + hp_ref[...]) + b_ref[...]
    t = jnp.where(t >= 0.0, t, 0.01 * t)
    o_ref[...] = dis * jnp.dot(t, w_ref[...],
                               preferred_element_type=jnp.float32)


def _fin_body(a0_ref, a1_ref, hp_ref, d0_ref, d1_ref, b_ref, o_ref):
    dis = _dis(d0_ref, d1_ref)
    o_ref[...] = dis * (a0_ref[...] + a1_ref[...] + hp_ref[...]) + b_ref[...]


_nd_spec = pl.BlockSpec((BM, D), lambda i: (i, 0))
_d_spec = pl.BlockSpec((BM, 1), lambda i: (i, 0))
_w_spec = pl.BlockSpec((D, D), lambda i: (0, 0))
_b_spec = pl.BlockSpec((1, D), lambda i: (0, 0))
_out_nd = jax.ShapeDtypeStruct((NP, D), jnp.float32)
_grid = (NP // BM,)

_mm1 = pl.pallas_call(
    _mm1_body, grid=_grid,
    in_specs=[_nd_spec, _w_spec, _d_spec, _d_spec],
    out_specs=_nd_spec, out_shape=_out_nd)

_mid = pl.pallas_call(
    _mid_body, grid=_grid,
    in_specs=[_nd_spec, _nd_spec, _nd_spec, _d_spec, _d_spec, _b_spec, _w_spec],
    out_specs=_nd_spec, out_shape=_out_nd)

_fin = pl.pallas_call(
    _fin_body, grid=(N // 1000,),
    in_specs=[pl.BlockSpec((1000, D), lambda i: (i, 0))] * 3 +
             [pl.BlockSpec((1000, 1), lambda i: (i, 0))] * 2 + [_b_spec],
    out_specs=pl.BlockSpec((1000, D), lambda i: (i, 0)),
    out_shape=jax.ShapeDtypeStruct((N, D), jnp.float32))


def kernel(x, edge_index, edge_attr, W1, b1, W2, b2, W3, b3):
    src = edge_index[0].astype(jnp.int32)
    dst = edge_index[1].astype(jnp.int32)
    ew = edge_attr.astype(jnp.float32)

    pad = E_PAD - E
    spread = (jnp.arange(pad, dtype=jnp.int32) * 13) % N
    srcm = jnp.concatenate([src, spread])
    dstm = jnp.concatenate([dst, spread])
    ewm = jnp.concatenate([ew, jnp.zeros((pad,), jnp.float32)])
    srcm = srcm.reshape(NW, NCHT, CH)
    dstm = dstm.reshape(NW, NCHT, CH)
    ewm = ewm.reshape(NW, NCHT, CH)

    xpad = jnp.concatenate([x, jnp.zeros((NP - N, D), jnp.float32)])
    z_n1 = jnp.zeros((NP,), jnp.float32)
    z_nd = jnp.zeros((NP, D), jnp.float32)

    ew_i = lax.bitcast_convert_type(ewm, jnp.int32)
    comb = jnp.concatenate([
        srcm.reshape(NW, NWIN, G, CH),
        dstm.reshape(NW, NWIN, G, CH),
        ew_i.reshape(NW, NWIN, G, CH),
    ], axis=2)

    degp = _deg_kernel(dstm, ewm, z_n1)
    d0 = degp[0].reshape(NP, 1)
    d1 = degp[1].reshape(NP, 1)

    b1r = b1.reshape(1, D)
    b2r = b2.reshape(1, D)
    b3r = b3.reshape(1, D)

    hp = _mm1(xpad, W1, d0, d1)
    accp = _agg_kernel(hp, comb, z_nd)
    hp = _mid(accp[0], accp[1], hp, d0, d1, b1r, W2)
    accp = _agg_kernel(hp, comb, z_nd)
    hp = _mid(accp[0], accp[1], hp, d0, d1, b2r, W3)
    accp = _agg_kernel(hp, comb, z_nd)
    return _fin(accp[0], accp[1], hp, d0, d1, b3r)
